# baseline (speedup 1.0000x reference)
import functools
import math

import jax
import jax.numpy as jnp
from jax import lax
from jax.experimental import pallas as pl
from jax.experimental.pallas import tpu as pltpu
from jax.experimental.pallas import tpu_sc as plsc

D_MODEL = 64
SCALE = math.sqrt(D_MODEL)
NC, NS = 2, 16
NW = NC * NS
SEQ = 50
NBATCH = 4096
B_ROWS = NBATCH * SEQ
BPW = B_ROWS // NW
BB = NBATCH // NW
MACRO = 640
NMACRO = BPW // MACRO


@jax.jit
def _sc_embed(x_t, lut):
    mesh = plsc.VectorSubcoreMesh(core_axis_name="c", subcore_axis_name="s")

    @functools.partial(
        pl.kernel,
        out_type=jax.ShapeDtypeStruct((B_ROWS, D_MODEL), jnp.float32),
        mesh=mesh,
        scratch_types=[
            pltpu.VMEM((SEQ, BB), jnp.int32),
            pltpu.VMEM((BPW,), jnp.int32),
            pltpu.VMEM((MACRO, D_MODEL), jnp.float32),
            pltpu.VMEM((MACRO, D_MODEL), jnp.float32),
            pltpu.SemaphoreType.DMA,
            pltpu.SemaphoreType.DMA,
            pltpu.SemaphoreType.DMA,
            pltpu.SemaphoreType.DMA,
        ],
        compiler_params=pltpu.CompilerParams(
            use_tc_tiling_on_sc=False, needs_layout_passes=False
        ),
    )
    def body(xt_hbm, lut_hbm, out_hbm, stage, idx_v, buf0, buf1, g0, g1, s0, s1):
        wid = lax.axis_index("s") * NC + lax.axis_index("c")

        for s in range(SEQ):
            pltpu.async_copy(
                xt_hbm.at[s, pl.ds(wid * BB, BB)], stage.at[s], g0
            )
        for s in range(SEQ):
            pltpu.make_async_copy(
                xt_hbm.at[s, pl.ds(wid * BB, BB)], stage.at[s], g0
            ).wait()

        lane_dst = lax.iota(jnp.int32, 16) * SEQ

        def t_body(s, c):
            for u in range(BB // 16):
                b0 = u * 16
                vals = stage[s, pl.ds(b0, 16)]
                dst = lane_dst + (b0 * SEQ + s)
                plsc.store_scatter(idx_v, [dst], vals)
            return c

        lax.fori_loop(0, SEQ, t_body, 0)

        GRP = 16
        UNROLL = 8
        NGRP = MACRO // GRP

        def fire_gathers(m, buf, sem):
            def g_body(g, c):
                for u in range(UNROLL):
                    off = g * (GRP * UNROLL) + u * GRP
                    iv = idx_v[pl.ds(m * MACRO + off, GRP)]
                    pltpu.async_copy(
                        lut_hbm.at[iv], buf.at[pl.ds(off, GRP)], sem
                    )
                return c

            lax.fori_loop(0, NGRP // UNROLL, g_body, 0)

        def drain_gathers(m, buf, sem):
            def g_body(g, c):
                for u in range(UNROLL):
                    off = g * (GRP * UNROLL) + u * GRP
                    iv = idx_v[pl.ds(m * MACRO + off, GRP)]
                    pltpu.make_async_copy(
                        lut_hbm.at[iv], buf.at[pl.ds(off, GRP)], sem
                    ).wait()
                return c

            lax.fori_loop(0, NGRP // UNROLL, g_body, 0)

        def fire_store(m, buf, sem):
            pltpu.async_copy(
                buf, out_hbm.at[pl.ds(wid * BPW + m * MACRO, MACRO)], sem
            )

        def drain_store(m, buf, sem):
            pltpu.make_async_copy(
                buf, out_hbm.at[pl.ds(wid * BPW + m * MACRO, MACRO)], sem
            ).wait()

        def scale(buf):
            @plsc.parallel_loop(0, MACRO, unroll=4)
            def _(r):
                for c in range(D_MODEL // 16):
                    sl = pl.ds(c * 16, 16)
                    buf[r, sl] = buf[r, sl] * SCALE

        fire_gathers(0, buf0, g0)

        def pair_body(p, carry):
            m0 = 2 * p
            m1 = m0 + 1
            drain_gathers(m0, buf0, g0)

            @pl.when(p > 0)
            def _():
                drain_store(m1 - 2, buf1, s1)

            fire_gathers(m1, buf1, g1)
            scale(buf0)
            fire_store(m0, buf0, s0)
            drain_gathers(m1, buf1, g1)
            drain_store(m0, buf0, s0)

            @pl.when(p < NMACRO // 2 - 1)
            def _():
                fire_gathers(m0 + 2, buf0, g0)

            scale(buf1)
            fire_store(m1, buf1, s1)
            return carry

        lax.fori_loop(0, NMACRO // 2, pair_body, 0)
        drain_store(NMACRO - 1, buf1, s1)

    return body(x_t, lut)


def kernel(x, lut):
    x_t = x.astype(jnp.int32).T
    out = _sc_embed(x_t, lut)
    return out.reshape(NBATCH, SEQ, D_MODEL)
